# baseline (speedup 1.0000x reference)
import functools

import jax
import jax.numpy as jnp
from jax import lax
from jax.experimental import pallas as pl
from jax.experimental.pallas import tpu as pltpu
from jax.experimental.pallas import tpu_sc as plsc

_N_NODES = 10000
_N_EDGES = 160000
_D = 256
_DH = 128
_NC = 2
_NS = 16
_B = 128
_NB = 80
_NBH = 40
_EDGES_PAD = _NS * _NB * _B
_ACC_ROWS = 10240
_ZROWS = _ACC_ROWS // _NS
_OROWS = _ACC_ROWS // _NS


def _sc_segment_sum(fr, src_r, dst_r, zeros):
    mesh = plsc.VectorSubcoreMesh(core_axis_name="c", subcore_axis_name="s")
    h_ty = jax.ShapeDtypeStruct((_ACC_ROWS, _DH), jnp.float32)

    @functools.partial(
        pl.kernel,
        out_type=[h_ty, h_ty],
        mesh=mesh,
        scratch_types=[
            pltpu.VMEM((_NBH, _B), jnp.int32),
            pltpu.VMEM((_NBH, _B), jnp.int32),
            pltpu.VMEM((_B, _DH), jnp.float32),
            pltpu.VMEM((_B, _DH), jnp.float32),
            pltpu.VMEM_SHARED((_ACC_ROWS, _DH), jnp.float32),
            pltpu.SemaphoreType.DMA,
            pltpu.SemaphoreType.DMA,
            pltpu.SemaphoreType.DMA,
            pltpu.SemaphoreType.DMA,
        ],
    )
    def scatter_kernel(f_hbm, src_hbm, dst_hbm, z_hbm,
                       h0_hbm, h1_hbm, src_v, dst_v, rows0, rows1, acc,
                       gsem0, gsem1, ssem0, ssem1):
        c = lax.axis_index("c")
        s = lax.axis_index("s")

        pltpu.sync_copy(z_hbm, acc.at[pl.ds(s * _ZROWS, _ZROWS)])
        plsc.subcore_barrier()

        def run():
            def g_start(i, buf, sem):
                pltpu.async_copy(f_hbm.at[src_v.at[i]], buf, sem)

            def g_wait(i, buf, sem):
                pltpu.make_async_copy(f_hbm.at[src_v.at[i]], buf, sem).wait()

            def s_start(i, buf, sem):
                pltpu.async_copy(buf, acc.at[dst_v.at[i]], sem, add=True)

            def s_wait(i, buf, sem):
                pltpu.make_async_copy(buf, acc.at[dst_v.at[i]], sem).wait()

            for half in range(_NB // _NBH):
                sl = pl.ds(half * _NBH, _NBH)
                pltpu.sync_copy(src_hbm.at[c, s, sl], src_v)
                pltpu.sync_copy(dst_hbm.at[s, sl], dst_v)

                g_start(0, rows0, gsem0)
                g_start(1, rows1, gsem1)
                g_wait(0, rows0, gsem0)
                s_start(0, rows0, ssem0)

                @pl.loop(0, (_NBH - 2) // 2)
                def _(j):
                    b = 2 * j
                    g_wait(b + 1, rows1, gsem1)
                    s_wait(b, rows0, ssem0)
                    g_start(b + 2, rows0, gsem0)
                    s_start(b + 1, rows1, ssem1)
                    g_wait(b + 2, rows0, gsem0)
                    s_wait(b + 1, rows1, ssem1)
                    g_start(b + 3, rows1, gsem1)
                    s_start(b + 2, rows0, ssem0)

                g_wait(_NBH - 1, rows1, gsem1)
                s_wait(_NBH - 2, rows0, ssem0)
                s_start(_NBH - 1, rows1, ssem1)
                s_wait(_NBH - 1, rows1, ssem1)

        run()

        plsc.subcore_barrier()

        def emit(h_hbm):
            sl = pl.ds(s * _OROWS, _OROWS)
            pltpu.sync_copy(acc.at[sl], h_hbm.at[sl])

        @pl.when(c == 0)
        def _():
            emit(h0_hbm)

        @pl.when(c == 1)
        def _():
            emit(h1_hbm)

    return scatter_kernel(fr, src_r, dst_r, zeros)


_MM_ROWS = 2000


def _mm_body(h0_ref, h1_ref, w_ref, b_ref, o_ref):
    acc = lax.dot_general(
        h0_ref[...], w_ref[:, :_DH],
        dimension_numbers=(((1,), (1,)), ((), ())),
        preferred_element_type=jnp.float32,
    )
    acc += lax.dot_general(
        h1_ref[...], w_ref[:, _DH:],
        dimension_numbers=(((1,), (1,)), ((), ())),
        preferred_element_type=jnp.float32,
    )
    o_ref[...] = acc + b_ref[...]


def _linear(h0, h1, W, b2):
    grid = (_N_NODES // _MM_ROWS,)
    return pl.pallas_call(
        _mm_body,
        grid=grid,
        in_specs=[
            pl.BlockSpec((_MM_ROWS, _DH), lambda i: (i, 0)),
            pl.BlockSpec((_MM_ROWS, _DH), lambda i: (i, 0)),
            pl.BlockSpec((_D, _D), lambda i: (0, 0)),
            pl.BlockSpec((1, _D), lambda i: (0, 0)),
        ],
        out_specs=pl.BlockSpec((_MM_ROWS, _D), lambda i: (i, 0)),
        out_shape=jax.ShapeDtypeStruct((_N_NODES, _D), jnp.float32),
    )(h0, h1, W, b2)


def kernel(feature, edge_index, W, b):
    fr = feature.reshape(2 * _N_NODES, _DH)
    ei = edge_index.astype(jnp.int32)
    pad = _EDGES_PAD - _N_EDGES
    src = jnp.concatenate([ei[0], jnp.zeros((pad,), jnp.int32)])
    dst = jnp.concatenate([ei[1], jnp.full((pad,), _N_NODES, jnp.int32)])
    src2 = 2 * src
    src_r = jnp.stack([src2, src2 + 1]).reshape(2, _NS, _NB, _B)
    dst_r = dst.reshape(_NS, _NB, _B)
    zeros = jnp.zeros((_ZROWS, _DH), jnp.float32)
    h0, h1 = _sc_segment_sum(fr, src_r, dst_r, zeros)
    return _linear(h0, h1, W, b.reshape(1, _D))
